# baseline (speedup 1.0000x reference)
import functools

import jax
import jax.numpy as jnp
from jax import lax
from jax.experimental import pallas as pl
from jax.experimental.pallas import tpu as pltpu
from jax.experimental.pallas import tpu_sc as plsc

N = 10000
E = 320000
D = 128
DO = 128

NC = 2
NS = 16
L = 16

K = 80
EPT = E // NS
NCHUNK = EPT // K
HALF = NCHUNK // 2
ROWS_PT = 640
ZCOPY = 80


def _sc_body(x_hbm, idx_hbm, attr_hbm, mi_hbm, mo_hbm,
             src_a, src_b, dst_a, dst_b, attr_a, attr_b, rows_a, rows_b, acc,
             gsem_a, gsem_b, asem_a, asem_b, sisem_a, sisem_b,
             disem_a, disem_b, ssem_a, ssem_b):
    cid = lax.axis_index("c")
    sid = lax.axis_index("s")
    ebase = sid * EPT

    src = (src_a, src_b)
    dst = (dst_a, dst_b)
    attr = (attr_a, attr_b)
    rows = (rows_a, rows_b)
    gsem = (gsem_a, gsem_b)
    asem = (asem_a, asem_b)
    sisem = (sisem_a, sisem_b)
    disem = (disem_a, disem_b)
    ssem = (ssem_a, ssem_b)

    def src_sl(g):
        return idx_hbm.at[pl.ds(cid * E + ebase + g * K, K)]

    def dst_sl(g):
        return idx_hbm.at[pl.ds((1 - cid) * E + ebase + g * K, K)]

    def attr_sl(g):
        return attr_hbm.at[pl.ds((ebase + g * K) * L, K * L)]

    def phase(g, t, s):
        o = 1 - s
        @pl.when(t + s > 0)
        def _():
            pltpu.make_async_copy(rows[o], acc.at[dst[o]], ssem[o]).wait()
        valid = (s == 0) or (t < HALF - 1)

        def prefetch():
            gn = 2 * t + 1 if s == 0 else 2 * t + 2
            pltpu.async_copy(src_sl(gn), src[o], sisem[o])
            pltpu.async_copy(dst_sl(gn), dst[o], disem[o])
            pltpu.async_copy(attr_sl(gn), attr[o], asem[o])
            pltpu.make_async_copy(src_sl(gn), src[o], sisem[o]).wait()
            pltpu.async_copy(x_hbm.at[src[o]], rows[o], gsem[o])
        if valid is True:
            prefetch()
        else:
            pl.when(t < HALF - 1)(prefetch)
        pltpu.make_async_copy(x_hbm.at[src[s]], rows[s], gsem[s]).wait()
        pltpu.make_async_copy(attr_sl(g), attr[s], asem[s]).wait()

        buf = rows[s]
        attr_buf = attr[s]

        def edge(k, _):
            a = attr_buf[pl.ds(k * L, L)]
            for j in range(D // L):
                buf[k, pl.ds(j * L, L)] = buf[k, pl.ds(j * L, L)] * a
            return 0
        lax.fori_loop(0, K, edge, 0)

        pltpu.make_async_copy(dst_sl(g), dst[s], disem[s]).wait()
        pltpu.async_copy(rows[s], acc.at[dst[s]], ssem[s], add=True)

    def zrow(r, _):
        for j in range(D // L):
            rows_a[r, pl.ds(j * L, L)] = jnp.zeros((L,), jnp.float32)
        return 0
    lax.fori_loop(0, K, zrow, 0)
    ncopies = jnp.where(sid == NS - 1, 5, 8)

    def zcopy(r, _):
        pltpu.sync_copy(rows_a, acc.at[pl.ds(sid * ROWS_PT + r * ZCOPY,
                                             ZCOPY), :])
        return 0
    lax.fori_loop(0, ncopies, zcopy, 0)
    plsc.subcore_barrier()

    pltpu.async_copy(src_sl(0), src_a, sisem_a)
    pltpu.async_copy(dst_sl(0), dst_a, disem_a)
    pltpu.async_copy(attr_sl(0), attr_a, asem_a)
    pltpu.make_async_copy(src_sl(0), src_a, sisem_a).wait()
    pltpu.async_copy(x_hbm.at[src_a], rows_a, gsem_a)

    def pipe(t, _):
        phase(2 * t, t, 0)
        phase(2 * t + 1, t, 1)
        return 0
    lax.fori_loop(0, HALF, pipe, 0)
    pltpu.make_async_copy(rows_b, acc.at[dst_b], ssem_b).wait()
    plsc.subcore_barrier()

    def wcopy(r, _):
        off = sid * ROWS_PT + r * ZCOPY

        @pl.when(cid == 0)
        def _():
            pltpu.sync_copy(acc.at[pl.ds(off, ZCOPY), :],
                            mi_hbm.at[pl.ds(off, ZCOPY), :])

        @pl.when(cid == 1)
        def _():
            pltpu.sync_copy(acc.at[pl.ds(off, ZCOPY), :],
                            mo_hbm.at[pl.ds(off, ZCOPY), :])
        return 0
    lax.fori_loop(0, ncopies, wcopy, 0)


_sc_scatter = functools.partial(
    pl.kernel,
    out_type=(jax.ShapeDtypeStruct((N, D), jnp.float32),
              jax.ShapeDtypeStruct((N, D), jnp.float32)),
    mesh=plsc.VectorSubcoreMesh(core_axis_name="c", subcore_axis_name="s",
                                num_cores=NC, num_subcores=NS),
    scratch_types=[
        pltpu.VMEM((K,), jnp.int32),
        pltpu.VMEM((K,), jnp.int32),
        pltpu.VMEM((K,), jnp.int32),
        pltpu.VMEM((K,), jnp.int32),
        pltpu.VMEM((K * L,), jnp.float32),
        pltpu.VMEM((K * L,), jnp.float32),
        pltpu.VMEM((K, D), jnp.float32),
        pltpu.VMEM((K, D), jnp.float32),
        pltpu.VMEM_SHARED((N, D), jnp.float32),
        pltpu.SemaphoreType.DMA,
        pltpu.SemaphoreType.DMA,
        pltpu.SemaphoreType.DMA,
        pltpu.SemaphoreType.DMA,
        pltpu.SemaphoreType.DMA,
        pltpu.SemaphoreType.DMA,
        pltpu.SemaphoreType.DMA,
        pltpu.SemaphoreType.DMA,
        pltpu.SemaphoreType.DMA,
        pltpu.SemaphoreType.DMA,
    ],
)(_sc_body)


def _mlp_body(mi_ref, mo_ref, x_ref, W1_ref, b1_ref, W2_ref, b2_ref, o_ref):
    acc = jnp.dot(mi_ref[...], W1_ref[0:D, :],
                  preferred_element_type=jnp.float32)
    acc += jnp.dot(mo_ref[...], W1_ref[D:2 * D, :],
                   preferred_element_type=jnp.float32)
    acc += jnp.dot(x_ref[...], W1_ref[2 * D:3 * D, :],
                   preferred_element_type=jnp.float32)
    h = jnp.tanh(acc + b1_ref[...])
    o_ref[...] = jnp.tanh(
        jnp.dot(h, W2_ref[...], preferred_element_type=jnp.float32)
        + b2_ref[...])


_BLK = 2000


def _mlp(mi, mo, x, W1, b1, W2, b2):
    grid = (N // _BLK,)
    return pl.pallas_call(
        _mlp_body,
        grid=grid,
        in_specs=[
            pl.BlockSpec((_BLK, D), lambda i: (i, 0)),
            pl.BlockSpec((_BLK, D), lambda i: (i, 0)),
            pl.BlockSpec((_BLK, D), lambda i: (i, 0)),
            pl.BlockSpec((3 * D, DO), lambda i: (0, 0)),
            pl.BlockSpec((1, DO), lambda i: (0, 0)),
            pl.BlockSpec((DO, DO), lambda i: (0, 0)),
            pl.BlockSpec((1, DO), lambda i: (0, 0)),
        ],
        out_specs=pl.BlockSpec((_BLK, DO), lambda i: (i, 0)),
        out_shape=jax.ShapeDtypeStruct((N, DO), jnp.float32),
    )(mi, mo, x, W1, b1, W2, b2)


@jax.jit
def kernel(x, edge_index, edge_attr, W1, b1, W2, b2):
    idxcat = edge_index.reshape(2 * E)
    attr16 = jnp.broadcast_to(edge_attr, (E, L)).reshape(E * L)
    mi, mo = _sc_scatter(x, idxcat, attr16)
    return _mlp(mi, mo, x, W1, b1.reshape(1, DO), W2,
                b2.reshape(1, DO))


# speedup vs baseline: 1.3094x; 1.3094x over previous
import functools

import jax
import jax.numpy as jnp
from jax import lax
from jax.experimental import pallas as pl
from jax.experimental.pallas import tpu as pltpu
from jax.experimental.pallas import tpu_sc as plsc

N = 10000
E = 320000
D = 128
DO = 128

NC = 2
NS = 16
L = 16

K = 80
EPT = E // NS
NCHUNK = EPT // K
HALF = NCHUNK // 2
ROWS_PT = 640
ZCOPY = 80


def _scale_rows(rows, attr):
    def edge(k, _):
        a = attr[k, :]
        for j in range(D // L):
            rows[k, pl.ds(j * L, L)] = rows[k, pl.ds(j * L, L)] * a
        return 0
    lax.fori_loop(0, K, edge, 0)


def _sc_body(x_hbm, row_hbm, col_hbm, attr_hbm, mi_hbm, mo_hbm,
             src_a, src_b, dst_a, dst_b, attr_a, attr_b, rows_a, rows_b, acc,
             gsem_a, gsem_b, asem_a, asem_b, sisem_a, sisem_b,
             disem_a, disem_b, ssem_a, ssem_b):
    cid = lax.axis_index("c")
    sid = lax.axis_index("s")
    ebase = sid * EPT

    src = (src_a, src_b)
    dst = (dst_a, dst_b)
    attr = (attr_a, attr_b)
    rows = (rows_a, rows_b)
    gsem = (gsem_a, gsem_b)
    asem = (asem_a, asem_b)
    sisem = (sisem_a, sisem_b)
    disem = (disem_a, disem_b)
    ssem = (ssem_a, ssem_b)

    def attr_slice(g):
        return attr_hbm.at[pl.ds(ebase + g * K, K), :]

    def issue_loads(g, s):
        @pl.when(cid == 0)
        def _():
            pltpu.async_copy(row_hbm.at[pl.ds(ebase + g * K, K)], src[s],
                             sisem[s])
            pltpu.async_copy(col_hbm.at[pl.ds(ebase + g * K, K)], dst[s],
                             disem[s])

        @pl.when(cid == 1)
        def _():
            pltpu.async_copy(col_hbm.at[pl.ds(ebase + g * K, K)], src[s],
                             sisem[s])
            pltpu.async_copy(row_hbm.at[pl.ds(ebase + g * K, K)], dst[s],
                             disem[s])
        pltpu.async_copy(attr_slice(g), attr[s], asem[s])
        pltpu.make_async_copy(row_hbm.at[pl.ds(ebase + g * K, K)], src[s],
                              sisem[s]).wait()
        pltpu.async_copy(x_hbm.at[src[s]], rows[s], gsem[s])

    def phase(g, t, s):
        o = 1 - s
        @pl.when(t + s > 0)
        def _():
            pltpu.make_async_copy(rows[o], acc.at[dst[o]], ssem[o]).wait()
        if s == 0:
            issue_loads(2 * t + 1, o)
        else:
            @pl.when(t < HALF - 1)
            def _():
                issue_loads(2 * t + 2, o)
        pltpu.make_async_copy(x_hbm.at[src[s]], rows[s], gsem[s]).wait()
        pltpu.make_async_copy(attr_slice(g), attr[s], asem[s]).wait()
        _scale_rows(rows[s], attr[s])
        pltpu.make_async_copy(col_hbm.at[pl.ds(ebase + g * K, K)], dst[s],
                              disem[s]).wait()
        pltpu.async_copy(rows[s], acc.at[dst[s]], ssem[s], add=True)

    def zrow(r, _):
        for j in range(D // L):
            rows_a[r, pl.ds(j * L, L)] = jnp.zeros((L,), jnp.float32)
        return 0
    lax.fori_loop(0, K, zrow, 0)
    ncopies = jnp.where(sid == NS - 1, 5, 8)

    def zcopy(r, _):
        pltpu.sync_copy(rows_a, acc.at[pl.ds(sid * ROWS_PT + r * ZCOPY,
                                             ZCOPY), :])
        return 0
    lax.fori_loop(0, ncopies, zcopy, 0)
    plsc.subcore_barrier()

    issue_loads(0, 0)

    def pipe(t, _):
        phase(2 * t, t, 0)
        phase(2 * t + 1, t, 1)
        return 0
    lax.fori_loop(0, HALF, pipe, 0)
    pltpu.make_async_copy(rows_b, acc.at[dst_b], ssem_b).wait()
    plsc.subcore_barrier()

    def wcopy(r, _):
        off = sid * ROWS_PT + r * ZCOPY

        @pl.when(cid == 0)
        def _():
            pltpu.sync_copy(acc.at[pl.ds(off, ZCOPY), :],
                            mi_hbm.at[pl.ds(off, ZCOPY), :])

        @pl.when(cid == 1)
        def _():
            pltpu.sync_copy(acc.at[pl.ds(off, ZCOPY), :],
                            mo_hbm.at[pl.ds(off, ZCOPY), :])
        return 0
    lax.fori_loop(0, ncopies, wcopy, 0)


_sc_scatter = functools.partial(
    pl.kernel,
    out_type=(jax.ShapeDtypeStruct((N, D), jnp.float32),
              jax.ShapeDtypeStruct((N, D), jnp.float32)),
    mesh=plsc.VectorSubcoreMesh(core_axis_name="c", subcore_axis_name="s",
                                num_cores=NC, num_subcores=NS),
    scratch_types=[
        pltpu.VMEM((K,), jnp.int32),
        pltpu.VMEM((K,), jnp.int32),
        pltpu.VMEM((K,), jnp.int32),
        pltpu.VMEM((K,), jnp.int32),
        pltpu.VMEM((K, L), jnp.float32),
        pltpu.VMEM((K, L), jnp.float32),
        pltpu.VMEM((K, D), jnp.float32),
        pltpu.VMEM((K, D), jnp.float32),
        pltpu.VMEM_SHARED((N, D), jnp.float32),
        pltpu.SemaphoreType.DMA,
        pltpu.SemaphoreType.DMA,
        pltpu.SemaphoreType.DMA,
        pltpu.SemaphoreType.DMA,
        pltpu.SemaphoreType.DMA,
        pltpu.SemaphoreType.DMA,
        pltpu.SemaphoreType.DMA,
        pltpu.SemaphoreType.DMA,
        pltpu.SemaphoreType.DMA,
        pltpu.SemaphoreType.DMA,
    ],
)(_sc_body)


def _mlp_body(mi_ref, mo_ref, x_ref, W1_ref, b1_ref, W2_ref, b2_ref, o_ref):
    acc = jnp.dot(mi_ref[...], W1_ref[0:D, :],
                  preferred_element_type=jnp.float32)
    acc += jnp.dot(mo_ref[...], W1_ref[D:2 * D, :],
                   preferred_element_type=jnp.float32)
    acc += jnp.dot(x_ref[...], W1_ref[2 * D:3 * D, :],
                   preferred_element_type=jnp.float32)
    h = jnp.tanh(acc + b1_ref[...])
    o_ref[...] = jnp.tanh(
        jnp.dot(h, W2_ref[...], preferred_element_type=jnp.float32)
        + b2_ref[...])


_BLK = 2000


def _mlp(mi, mo, x, W1, b1, W2, b2):
    grid = (N // _BLK,)
    return pl.pallas_call(
        _mlp_body,
        grid=grid,
        in_specs=[
            pl.BlockSpec((_BLK, D), lambda i: (i, 0)),
            pl.BlockSpec((_BLK, D), lambda i: (i, 0)),
            pl.BlockSpec((_BLK, D), lambda i: (i, 0)),
            pl.BlockSpec((3 * D, DO), lambda i: (0, 0)),
            pl.BlockSpec((1, DO), lambda i: (0, 0)),
            pl.BlockSpec((DO, DO), lambda i: (0, 0)),
            pl.BlockSpec((1, DO), lambda i: (0, 0)),
        ],
        out_specs=pl.BlockSpec((_BLK, DO), lambda i: (i, 0)),
        out_shape=jax.ShapeDtypeStruct((N, DO), jnp.float32),
    )(mi, mo, x, W1, b1, W2, b2)


@jax.jit
def kernel(x, edge_index, edge_attr, W1, b1, W2, b2):
    row = edge_index[0]
    col = edge_index[1]
    attr16 = jnp.broadcast_to(edge_attr, (E, L))
    mi, mo = _sc_scatter(x, row, col, attr16)
    return _mlp(mi, mo, x, W1, b1.reshape(1, DO), W2, b2.reshape(1, DO))


# speedup vs baseline: 1.4298x; 1.0919x over previous
import functools

import jax
import jax.numpy as jnp
from jax import lax
from jax.experimental import pallas as pl
from jax.experimental.pallas import tpu as pltpu
from jax.experimental.pallas import tpu_sc as plsc

N = 10000
E = 320000
D = 128
DO = 128

NC = 2
NS = 16
L = 16

K = 128
NCH = E // K
BASECH = NCH // NS
HALF = BASECH // 2
AR = K * L // D
ROWS_PT = 640
ZCOPY = 80


def _scale_rows(rows, attr):
    def edge(k, _):
        a = attr[k // 8, pl.ds((k % 8) * L, L)]
        for j in range(D // L):
            rows[k, pl.ds(j * L, L)] = rows[k, pl.ds(j * L, L)] * a
        return 0
    lax.fori_loop(0, K, edge, 0)


def _sc_body(x_hbm, row_hbm, col_hbm, attr_hbm, mi_hbm, mo_hbm,
             src_a, src_b, dst_a, dst_b, attr_a, attr_b, rows_a, rows_b, acc,
             gsem_a, gsem_b, asem_a, asem_b, sisem_a, sisem_b,
             disem_a, disem_b, ssem_a, ssem_b):
    cid = lax.axis_index("c")
    sid = lax.axis_index("s")
    chbase = sid * BASECH + jnp.minimum(sid, 4)

    src = (src_a, src_b)
    dst = (dst_a, dst_b)
    attr = (attr_a, attr_b)
    rows = (rows_a, rows_b)
    gsem = (gsem_a, gsem_b)
    asem = (asem_a, asem_b)
    sisem = (sisem_a, sisem_b)
    disem = (disem_a, disem_b)
    ssem = (ssem_a, ssem_b)

    def attr_slice(g):
        return attr_hbm.at[pl.ds((chbase + g) * AR, AR), :]

    def issue_loads(g, s):
        @pl.when(cid == 0)
        def _():
            pltpu.async_copy(row_hbm.at[pl.ds((chbase + g) * K, K)], src[s],
                             sisem[s])
            pltpu.async_copy(col_hbm.at[pl.ds((chbase + g) * K, K)], dst[s],
                             disem[s])

        @pl.when(cid == 1)
        def _():
            pltpu.async_copy(col_hbm.at[pl.ds((chbase + g) * K, K)], src[s],
                             sisem[s])
            pltpu.async_copy(row_hbm.at[pl.ds((chbase + g) * K, K)], dst[s],
                             disem[s])
        pltpu.async_copy(attr_slice(g), attr[s], asem[s])
        pltpu.make_async_copy(row_hbm.at[pl.ds((chbase + g) * K, K)], src[s],
                              sisem[s]).wait()
        pltpu.async_copy(x_hbm.at[src[s]], rows[s], gsem[s])

    def phase(g, t, s):
        o = 1 - s
        @pl.when(t + s > 0)
        def _():
            pltpu.make_async_copy(rows[o], acc.at[dst[o]], ssem[o]).wait()
        if s == 0:
            issue_loads(2 * t + 1, o)
        else:
            @pl.when(t < HALF - 1)
            def _():
                issue_loads(2 * t + 2, o)

            @pl.when(jnp.logical_and(t == HALF - 1, sid < 4))
            def _():
                issue_loads(2 * t + 2, o)
        pltpu.make_async_copy(x_hbm.at[src[s]], rows[s], gsem[s]).wait()
        pltpu.make_async_copy(attr_slice(g), attr[s], asem[s]).wait()
        _scale_rows(rows[s], attr[s])
        pltpu.make_async_copy(col_hbm.at[pl.ds((chbase + g) * K, K)], dst[s],
                              disem[s]).wait()
        pltpu.async_copy(rows[s], acc.at[dst[s]], ssem[s], add=True)

    def zrow(r, _):
        for j in range(D // L):
            rows_a[r, pl.ds(j * L, L)] = jnp.zeros((L,), jnp.float32)
        return 0
    lax.fori_loop(0, K, zrow, 0)
    ncopies = jnp.where(sid == NS - 1, 5, 8)

    def zcopy(r, _):
        pltpu.sync_copy(rows_a.at[pl.ds(0, ZCOPY), :],
                        acc.at[pl.ds(sid * ROWS_PT + r * ZCOPY, ZCOPY), :])
        return 0
    lax.fori_loop(0, ncopies, zcopy, 0)
    plsc.subcore_barrier()

    issue_loads(0, 0)

    def pipe(t, _):
        phase(2 * t, t, 0)
        phase(2 * t + 1, t, 1)
        return 0
    lax.fori_loop(0, HALF, pipe, 0)
    pltpu.make_async_copy(rows_b, acc.at[dst_b], ssem_b).wait()

    @pl.when(sid < 4)
    def _():
        g = BASECH
        pltpu.make_async_copy(x_hbm.at[src_a], rows_a, gsem_a).wait()
        pltpu.make_async_copy(attr_slice(g), attr_a, asem_a).wait()
        _scale_rows(rows_a, attr_a)
        pltpu.make_async_copy(col_hbm.at[pl.ds((chbase + g) * K, K)], dst_a,
                              disem_a).wait()
        pltpu.async_copy(rows_a, acc.at[dst_a], ssem_a, add=True)
        pltpu.make_async_copy(rows_a, acc.at[dst_a], ssem_a).wait()
    plsc.subcore_barrier()

    def wcopy(r, _):
        off = sid * ROWS_PT + r * ZCOPY

        @pl.when(cid == 0)
        def _():
            pltpu.sync_copy(acc.at[pl.ds(off, ZCOPY), :],
                            mi_hbm.at[pl.ds(off, ZCOPY), :])

        @pl.when(cid == 1)
        def _():
            pltpu.sync_copy(acc.at[pl.ds(off, ZCOPY), :],
                            mo_hbm.at[pl.ds(off, ZCOPY), :])
        return 0
    lax.fori_loop(0, ncopies, wcopy, 0)


_sc_scatter = functools.partial(
    pl.kernel,
    out_type=(jax.ShapeDtypeStruct((N, D), jnp.float32),
              jax.ShapeDtypeStruct((N, D), jnp.float32)),
    mesh=plsc.VectorSubcoreMesh(core_axis_name="c", subcore_axis_name="s",
                                num_cores=NC, num_subcores=NS),
    scratch_types=[
        pltpu.VMEM((K,), jnp.int32),
        pltpu.VMEM((K,), jnp.int32),
        pltpu.VMEM((K,), jnp.int32),
        pltpu.VMEM((K,), jnp.int32),
        pltpu.VMEM((AR, D), jnp.float32),
        pltpu.VMEM((AR, D), jnp.float32),
        pltpu.VMEM((K, D), jnp.float32),
        pltpu.VMEM((K, D), jnp.float32),
        pltpu.VMEM_SHARED((N, D), jnp.float32),
        pltpu.SemaphoreType.DMA,
        pltpu.SemaphoreType.DMA,
        pltpu.SemaphoreType.DMA,
        pltpu.SemaphoreType.DMA,
        pltpu.SemaphoreType.DMA,
        pltpu.SemaphoreType.DMA,
        pltpu.SemaphoreType.DMA,
        pltpu.SemaphoreType.DMA,
        pltpu.SemaphoreType.DMA,
        pltpu.SemaphoreType.DMA,
    ],
)(_sc_body)


def _mlp_body(mi_ref, mo_ref, x_ref, W1_ref, b1_ref, W2_ref, b2_ref, o_ref):
    acc = jnp.dot(mi_ref[...], W1_ref[0:D, :],
                  preferred_element_type=jnp.float32)
    acc += jnp.dot(mo_ref[...], W1_ref[D:2 * D, :],
                   preferred_element_type=jnp.float32)
    acc += jnp.dot(x_ref[...], W1_ref[2 * D:3 * D, :],
                   preferred_element_type=jnp.float32)
    h = jnp.tanh(acc + b1_ref[...])
    o_ref[...] = jnp.tanh(
        jnp.dot(h, W2_ref[...], preferred_element_type=jnp.float32)
        + b2_ref[...])


_BLK = 2000


def _mlp(mi, mo, x, W1, b1, W2, b2):
    grid = (N // _BLK,)
    return pl.pallas_call(
        _mlp_body,
        grid=grid,
        in_specs=[
            pl.BlockSpec((_BLK, D), lambda i: (i, 0)),
            pl.BlockSpec((_BLK, D), lambda i: (i, 0)),
            pl.BlockSpec((_BLK, D), lambda i: (i, 0)),
            pl.BlockSpec((3 * D, DO), lambda i: (0, 0)),
            pl.BlockSpec((1, DO), lambda i: (0, 0)),
            pl.BlockSpec((DO, DO), lambda i: (0, 0)),
            pl.BlockSpec((1, DO), lambda i: (0, 0)),
        ],
        out_specs=pl.BlockSpec((_BLK, DO), lambda i: (i, 0)),
        out_shape=jax.ShapeDtypeStruct((N, DO), jnp.float32),
    )(mi, mo, x, W1, b1, W2, b2)


@jax.jit
def kernel(x, edge_index, edge_attr, W1, b1, W2, b2):
    row = edge_index[0]
    col = edge_index[1]
    attr16 = jnp.broadcast_to(edge_attr, (E, L)).reshape(E * L // D, D)
    mi, mo = _sc_scatter(x, row, col, attr16)
    return _mlp(mi, mo, x, W1, b1.reshape(1, DO), W2, b2.reshape(1, DO))


# speedup vs baseline: 1.7021x; 1.1905x over previous
import functools

import jax
import jax.numpy as jnp
from jax import lax
from jax.experimental import pallas as pl
from jax.experimental.pallas import tpu as pltpu
from jax.experimental.pallas import tpu_sc as plsc

N = 10000
E = 320000
D = 128
DO = 128

NC = 2
NS = 16
L = 16

K = 128
NCH = E // K
BASECH = NCH // NS
HALF = BASECH // 2
AR = K * L // D
ROWS_PT = 640
ZCOPY = 80


def _scale_rows(rows, attr):
    @plsc.parallel_loop(0, K, 1, unroll=2)
    def _(k):
        a = attr[k // 8, pl.ds((k % 8) * L, L)]
        for j in range(D // L):
            rows[k, pl.ds(j * L, L)] = rows[k, pl.ds(j * L, L)] * a


def _sc_body(x_hbm, row_hbm, col_hbm, attr_hbm, mi_hbm, mo_hbm,
             src_a, src_b, dst_a, dst_b, attr_a, attr_b, rows_a, rows_b, acc,
             gsem_a, gsem_b, asem_a, asem_b, sisem_a, sisem_b,
             disem_a, disem_b, ssem_a, ssem_b):
    cid = lax.axis_index("c")
    sid = lax.axis_index("s")
    chbase = sid * BASECH + jnp.minimum(sid, 4)

    src = (src_a, src_b)
    dst = (dst_a, dst_b)
    attr = (attr_a, attr_b)
    rows = (rows_a, rows_b)
    gsem = (gsem_a, gsem_b)
    asem = (asem_a, asem_b)
    sisem = (sisem_a, sisem_b)
    disem = (disem_a, disem_b)
    ssem = (ssem_a, ssem_b)

    def attr_slice(g):
        return attr_hbm.at[pl.ds((chbase + g) * AR, AR), :]

    def issue_loads(g, s):
        @pl.when(cid == 0)
        def _():
            pltpu.async_copy(row_hbm.at[pl.ds((chbase + g) * K, K)], src[s],
                             sisem[s])
            pltpu.async_copy(col_hbm.at[pl.ds((chbase + g) * K, K)], dst[s],
                             disem[s])

        @pl.when(cid == 1)
        def _():
            pltpu.async_copy(col_hbm.at[pl.ds((chbase + g) * K, K)], src[s],
                             sisem[s])
            pltpu.async_copy(row_hbm.at[pl.ds((chbase + g) * K, K)], dst[s],
                             disem[s])
        pltpu.async_copy(attr_slice(g), attr[s], asem[s])
        pltpu.make_async_copy(row_hbm.at[pl.ds((chbase + g) * K, K)], src[s],
                              sisem[s]).wait()
        pltpu.async_copy(x_hbm.at[src[s]], rows[s], gsem[s])

    def phase(g, t, s):
        o = 1 - s
        @pl.when(t + s > 0)
        def _():
            pltpu.make_async_copy(rows[o], acc.at[dst[o]], ssem[o]).wait()
        if s == 0:
            issue_loads(2 * t + 1, o)
        else:
            @pl.when(t < HALF - 1)
            def _():
                issue_loads(2 * t + 2, o)

            @pl.when(jnp.logical_and(t == HALF - 1, sid < 4))
            def _():
                issue_loads(2 * t + 2, o)
        pltpu.make_async_copy(x_hbm.at[src[s]], rows[s], gsem[s]).wait()
        pltpu.make_async_copy(attr_slice(g), attr[s], asem[s]).wait()
        _scale_rows(rows[s], attr[s])
        pltpu.make_async_copy(col_hbm.at[pl.ds((chbase + g) * K, K)], dst[s],
                              disem[s]).wait()
        pltpu.async_copy(rows[s], acc.at[dst[s]], ssem[s], add=True)

    def zrow(r, _):
        for j in range(D // L):
            rows_a[r, pl.ds(j * L, L)] = jnp.zeros((L,), jnp.float32)
        return 0
    lax.fori_loop(0, K, zrow, 0)
    ncopies = jnp.where(sid == NS - 1, 5, 8)

    def zcopy(r, _):
        pltpu.sync_copy(rows_a.at[pl.ds(0, ZCOPY), :],
                        acc.at[pl.ds(sid * ROWS_PT + r * ZCOPY, ZCOPY), :])
        return 0
    lax.fori_loop(0, ncopies, zcopy, 0)
    plsc.subcore_barrier()

    issue_loads(0, 0)

    def pipe(t, _):
        phase(2 * t, t, 0)
        phase(2 * t + 1, t, 1)
        return 0
    lax.fori_loop(0, HALF, pipe, 0)
    pltpu.make_async_copy(rows_b, acc.at[dst_b], ssem_b).wait()

    @pl.when(sid < 4)
    def _():
        g = BASECH
        pltpu.make_async_copy(x_hbm.at[src_a], rows_a, gsem_a).wait()
        pltpu.make_async_copy(attr_slice(g), attr_a, asem_a).wait()
        _scale_rows(rows_a, attr_a)
        pltpu.make_async_copy(col_hbm.at[pl.ds((chbase + g) * K, K)], dst_a,
                              disem_a).wait()
        pltpu.async_copy(rows_a, acc.at[dst_a], ssem_a, add=True)
        pltpu.make_async_copy(rows_a, acc.at[dst_a], ssem_a).wait()
    plsc.subcore_barrier()

    def wcopy(r, _):
        off = sid * ROWS_PT + r * ZCOPY

        @pl.when(cid == 0)
        def _():
            pltpu.sync_copy(acc.at[pl.ds(off, ZCOPY), :],
                            mi_hbm.at[pl.ds(off, ZCOPY), :])

        @pl.when(cid == 1)
        def _():
            pltpu.sync_copy(acc.at[pl.ds(off, ZCOPY), :],
                            mo_hbm.at[pl.ds(off, ZCOPY), :])
        return 0
    lax.fori_loop(0, ncopies, wcopy, 0)


_sc_scatter = functools.partial(
    pl.kernel,
    out_type=(jax.ShapeDtypeStruct((N, D), jnp.float32),
              jax.ShapeDtypeStruct((N, D), jnp.float32)),
    mesh=plsc.VectorSubcoreMesh(core_axis_name="c", subcore_axis_name="s",
                                num_cores=NC, num_subcores=NS),
    scratch_types=[
        pltpu.VMEM((K,), jnp.int32),
        pltpu.VMEM((K,), jnp.int32),
        pltpu.VMEM((K,), jnp.int32),
        pltpu.VMEM((K,), jnp.int32),
        pltpu.VMEM((AR, D), jnp.float32),
        pltpu.VMEM((AR, D), jnp.float32),
        pltpu.VMEM((K, D), jnp.float32),
        pltpu.VMEM((K, D), jnp.float32),
        pltpu.VMEM_SHARED((N, D), jnp.float32),
        pltpu.SemaphoreType.DMA,
        pltpu.SemaphoreType.DMA,
        pltpu.SemaphoreType.DMA,
        pltpu.SemaphoreType.DMA,
        pltpu.SemaphoreType.DMA,
        pltpu.SemaphoreType.DMA,
        pltpu.SemaphoreType.DMA,
        pltpu.SemaphoreType.DMA,
        pltpu.SemaphoreType.DMA,
        pltpu.SemaphoreType.DMA,
    ],
)(_sc_body)


def _mlp_body(mi_ref, mo_ref, x_ref, W1_ref, b1_ref, W2_ref, b2_ref, o_ref):
    acc = jnp.dot(mi_ref[...], W1_ref[0:D, :],
                  preferred_element_type=jnp.float32)
    acc += jnp.dot(mo_ref[...], W1_ref[D:2 * D, :],
                   preferred_element_type=jnp.float32)
    acc += jnp.dot(x_ref[...], W1_ref[2 * D:3 * D, :],
                   preferred_element_type=jnp.float32)
    h = jnp.tanh(acc + b1_ref[...])
    o_ref[...] = jnp.tanh(
        jnp.dot(h, W2_ref[...], preferred_element_type=jnp.float32)
        + b2_ref[...])


_BLK = 2000


def _mlp(mi, mo, x, W1, b1, W2, b2):
    grid = (N // _BLK,)
    return pl.pallas_call(
        _mlp_body,
        grid=grid,
        in_specs=[
            pl.BlockSpec((_BLK, D), lambda i: (i, 0)),
            pl.BlockSpec((_BLK, D), lambda i: (i, 0)),
            pl.BlockSpec((_BLK, D), lambda i: (i, 0)),
            pl.BlockSpec((3 * D, DO), lambda i: (0, 0)),
            pl.BlockSpec((1, DO), lambda i: (0, 0)),
            pl.BlockSpec((DO, DO), lambda i: (0, 0)),
            pl.BlockSpec((1, DO), lambda i: (0, 0)),
        ],
        out_specs=pl.BlockSpec((_BLK, DO), lambda i: (i, 0)),
        out_shape=jax.ShapeDtypeStruct((N, DO), jnp.float32),
    )(mi, mo, x, W1, b1, W2, b2)


@jax.jit
def kernel(x, edge_index, edge_attr, W1, b1, W2, b2):
    row = edge_index[0]
    col = edge_index[1]
    attr16 = jnp.broadcast_to(edge_attr, (E, L)).reshape(E * L // D, D)
    mi, mo = _sc_scatter(x, row, col, attr16)
    return _mlp(mi, mo, x, W1, b1.reshape(1, DO), W2, b2.reshape(1, DO))
